# Initial kernel scaffold; baseline (speedup 1.0000x reference)
import jax
import jax.numpy as jnp
from jax.experimental import pallas as pl


def kernel(x, adj, w_enc, w_ext, omega, beta, w_dec):
    raise NotImplementedError("write your pallas kernel here")


# speedup vs baseline: 12.3395x; 12.3395x over previous
import functools

import jax
import jax.numpy as jnp
from jax import lax
from jax.experimental import pallas as pl
from jax.experimental.pallas import tpu as pltpu
from jax.experimental.pallas import tpu_sc as plsc

N = 10000
E = 320000
NFEAT = 128
NHID = 128
NCLASS = 16
STEP = 0.1

NC = 2
NS = 16
NW = NC * NS

LANE = 128
CH = 79
EPT = CH * LANE
EPAD = NW * EPT
NPAD = 10240
RPT = NPAD // NS

_mesh = plsc.VectorSubcoreMesh(core_axis_name="c", subcore_axis_name="s")


@functools.partial(
    pl.kernel,
    out_type=jax.ShapeDtypeStruct((NC, NPAD), jnp.float32),
    mesh=_mesh,
    scratch_types=[
        pltpu.VMEM((CH, LANE), jnp.int32),
        pltpu.VMEM((RPT,), jnp.float32),
        pltpu.VMEM((LANE,), jnp.float32),
        pltpu.VMEM_SHARED((NPAD,), jnp.float32),
    ],
)
def _deg_kernel(dst_hbm, out_hbm, dst_v, zero_v, ones_v, deg_sh):
    c = lax.axis_index("c")
    s = lax.axis_index("s")
    gid = c * NS + s

    def fill(i, _):
        zero_v[pl.ds(i * 16, 16)] = jnp.zeros((16,), jnp.float32)
        return _

    lax.fori_loop(0, RPT // 16, fill, 0)
    for j in range(LANE // 16):
        ones_v[pl.ds(j * 16, 16)] = jnp.full((16,), 1.0, jnp.float32)

    pltpu.sync_copy(zero_v, deg_sh.at[pl.ds(s * RPT, RPT)])
    plsc.subcore_barrier()

    pltpu.sync_copy(dst_hbm.at[gid], dst_v)

    def step(ch, _):
        pltpu.sync_copy(ones_v, deg_sh.at[dst_v.at[ch]], add=True)
        return _

    lax.fori_loop(0, CH, step, 0)
    plsc.subcore_barrier()
    pltpu.sync_copy(deg_sh.at[pl.ds(s * RPT, RPT)],
                    out_hbm.at[c, pl.ds(s * RPT, RPT)])


@functools.partial(
    pl.kernel,
    out_type=jax.ShapeDtypeStruct((NC, NPAD, NHID), jnp.float32),
    mesh=_mesh,
    scratch_types=[
        pltpu.VMEM((CH, LANE), jnp.int32),
        pltpu.VMEM((CH, LANE), jnp.int32),
        pltpu.VMEM((LANE, NHID), jnp.float32),
        pltpu.VMEM_SHARED((NPAD, NHID), jnp.float32),
        pltpu.SemaphoreType.DMA,
    ],
)
def _spmm_kernel(table_hbm, src_hbm, dst_hbm, out_hbm,
                 src_v, dst_v, rows_v, acc_sh, gsem):
    c = lax.axis_index("c")
    s = lax.axis_index("s")
    gid = c * NS + s

    def fill(i, _):
        r = i // (NHID // 16)
        j = i - r * (NHID // 16)
        rows_v[r, pl.ds(j * 16, 16)] = jnp.zeros((16,), jnp.float32)
        return _

    lax.fori_loop(0, LANE * (NHID // 16), fill, 0)
    for b in range(RPT // LANE):
        pltpu.sync_copy(rows_v, acc_sh.at[pl.ds(s * RPT + b * LANE, LANE)])
    plsc.subcore_barrier()

    pltpu.sync_copy(src_hbm.at[gid], src_v)
    pltpu.sync_copy(dst_hbm.at[gid], dst_v)

    def step(ch, _):
        pltpu.async_copy(table_hbm.at[src_v.at[ch]], rows_v, gsem).wait()
        pltpu.sync_copy(rows_v, acc_sh.at[dst_v.at[ch]], add=True)
        return _

    lax.fori_loop(0, CH, step, 0)
    plsc.subcore_barrier()
    for b in range(RPT // LANE):
        r = s * RPT + b * LANE
        pltpu.sync_copy(acc_sh.at[pl.ds(r, LANE)], out_hbm.at[c, pl.ds(r, LANE)])


BLK = 2048


def _enc_body(x_ref, we_ref, om_ref, deg_ref, h_ref, mp_ref, dinv_ref):
    h = jnp.dot(x_ref[...], we_ref[...], preferred_element_type=jnp.float32)
    d = deg_ref[...]
    dinv = lax.rsqrt(d[:, 0:1] + d[:, 1:2] + 1.0)
    om = om_ref[...]
    wsym = 0.5 * (om + om.T)
    m = jnp.dot(h, wsym, preferred_element_type=jnp.float32)
    h_ref[...] = h
    mp_ref[...] = dinv * m
    dinv_ref[...] = dinv


def _layer_body(h_ref, h0_ref, p_ref, mp_ref, dinv_ref, wext_ref, beta_ref,
                om_ref, hn_ref, mpn_ref):
    h = h_ref[...]
    mp = mp_ref[...]
    dinv = dinv_ref[...]
    agg = dinv * (p_ref[0] + p_ref[1] + mp)
    dh = agg - h * wext_ref[...] - beta_ref[0, 0] * h0_ref[...]
    hn = h + STEP * dh
    om = om_ref[...]
    wsym = 0.5 * (om + om.T)
    hn_ref[...] = hn
    mpn_ref[...] = dinv * jnp.dot(hn, wsym, preferred_element_type=jnp.float32)


def _final_body(h_ref, h0_ref, p_ref, mp_ref, dinv_ref, wext_ref, beta_ref,
                wd_ref, out_ref):
    h = h_ref[...]
    agg = dinv_ref[...] * (p_ref[0] + p_ref[1] + mp_ref[...])
    dh = agg - h * wext_ref[...] - beta_ref[0, 0] * h0_ref[...]
    hn = h + STEP * dh
    o = jnp.dot(hn, wd_ref[...], preferred_element_type=jnp.float32)
    m = jnp.max(o, axis=1, keepdims=True)
    e = o - m
    lse = jnp.log(jnp.sum(jnp.exp(e), axis=1, keepdims=True))
    out_ref[...] = e - lse


def _rows(i):
    return (i, 0)


def _fixed(i):
    return (0, 0)


def _rows_spec(w):
    return pl.BlockSpec((BLK, w), _rows)


def _full_spec(a, b):
    return pl.BlockSpec((a, b), _fixed)


_GRID = NPAD // BLK


def _enc_call(xp, w_enc, omega, degp_t):
    return pl.pallas_call(
        _enc_body,
        grid=(_GRID,),
        in_specs=[_rows_spec(NFEAT), _full_spec(NFEAT, NHID),
                  _full_spec(NHID, NHID), _rows_spec(2)],
        out_specs=[_rows_spec(NHID), _rows_spec(NHID), _rows_spec(1)],
        out_shape=[jax.ShapeDtypeStruct((NPAD, NHID), jnp.float32),
                   jax.ShapeDtypeStruct((NPAD, NHID), jnp.float32),
                   jax.ShapeDtypeStruct((NPAD, 1), jnp.float32)],
    )(xp, w_enc, omega, degp_t)


def _layer_call(h, h0, parts, mp, dinv, wext2, beta2, omega):
    pspec = pl.BlockSpec((NC, BLK, NHID), lambda i: (0, i, 0))
    return pl.pallas_call(
        _layer_body,
        grid=(_GRID,),
        in_specs=[_rows_spec(NHID), _rows_spec(NHID), pspec, _rows_spec(NHID),
                  _rows_spec(1), _full_spec(1, NHID), _full_spec(1, 1),
                  _full_spec(NHID, NHID)],
        out_specs=[_rows_spec(NHID), _rows_spec(NHID)],
        out_shape=[jax.ShapeDtypeStruct((NPAD, NHID), jnp.float32),
                   jax.ShapeDtypeStruct((NPAD, NHID), jnp.float32)],
    )(h, h0, parts, mp, dinv, wext2, beta2, omega)


def _final_call(h, h0, parts, mp, dinv, wext2, beta2, w_dec):
    pspec = pl.BlockSpec((NC, BLK, NHID), lambda i: (0, i, 0))
    return pl.pallas_call(
        _final_body,
        grid=(_GRID,),
        in_specs=[_rows_spec(NHID), _rows_spec(NHID), pspec, _rows_spec(NHID),
                  _rows_spec(1), _full_spec(1, NHID), _full_spec(1, 1),
                  _full_spec(NHID, NCLASS)],
        out_specs=_rows_spec(NCLASS),
        out_shape=jax.ShapeDtypeStruct((NPAD, NCLASS), jnp.float32),
    )(h, h0, parts, mp, dinv, wext2, beta2, w_dec)


@jax.jit
def kernel(x, adj, w_enc, w_ext, omega, beta, w_dec):
    src = adj[0]
    dst = adj[1]
    pad = jnp.full((EPAD - E,), N, jnp.int32)
    src_p = jnp.concatenate([src, pad]).reshape(NW, CH, LANE)
    dst_p = jnp.concatenate([dst, pad]).reshape(NW, CH, LANE)
    xp = jnp.zeros((NPAD, NFEAT), jnp.float32).at[:N].set(x)
    wext2 = w_ext.reshape(1, NHID)
    beta2 = beta.reshape(1, 1)

    degp = _deg_kernel(dst_p)
    h, mp1, dinv = _enc_call(xp, w_enc, omega, degp.T)
    parts1 = _spmm_kernel(mp1, src_p, dst_p)
    h1, mp2 = _layer_call(h, h, parts1, mp1, dinv, wext2, beta2, omega)
    parts2 = _spmm_kernel(mp2, src_p, dst_p)
    out = _final_call(h1, h, parts2, mp2, dinv, wext2, beta2, w_dec)
    return out[:N]
